# baseline (speedup 1.0000x reference)
import jax
import jax.numpy as jnp
from jax.experimental import pallas as pl

_B = 1024
_D = 768
_T = 2
_G = 4
_E = 8
_TOPK = 2
_HP = jax.lax.Precision.DEFAULT


def _mlp(x, mlp):
    n = len(mlp)
    h = x
    for i, (W, b) in enumerate(mlp):
        h = jnp.dot(h, W, precision=_HP) + b
        if i < n - 1:
            h = jax.nn.relu(h)
    return h


def _router(expert_in, gate_in, p):
    logits = jnp.dot(gate_in, p["gate_W"], precision=_HP) + p["gate_b"]
    topv, topi = jax.lax.top_k(logits, _TOPK)
    w = jax.nn.softmax(topv, axis=1)
    nb = logits.shape[0]
    dense = jnp.zeros_like(logits)
    dense = dense.at[jnp.arange(nb)[:, None], topi].set(w)
    outs = jax.vmap(lambda mlp: _mlp(expert_in, mlp))(p["mlp"])
    return jnp.einsum('be,ebd->bd', dense, outs, precision=_HP)


def _passthrough_kernel(x_ref, o_ref):
    o_ref[...] = x_ref[...]


def kernel(embeddings, params):
    group_outs = []
    for g in range(_G):
        xg = embeddings[:, g * _T:(g + 1) * _T, :]
        gate_in = xg.reshape(xg.shape[0], _T * _D)
        expert_in = xg.mean(axis=1)
        group_outs.append(_router(expert_in, gate_in, params["groups"][g]))
    stacked = jnp.stack(group_outs, axis=1)
    clf_gate_in = stacked.reshape(stacked.shape[0], _G * _D)
    clf_expert_in = stacked.mean(axis=1)
    logits = _router(clf_expert_in, clf_gate_in, params["clf"])
    return pl.pallas_call(
        _passthrough_kernel,
        out_shape=jax.ShapeDtypeStruct(logits.shape, logits.dtype),
    )(logits)


# speedup vs baseline: 1.4554x; 1.4554x over previous
import functools

import jax
import jax.numpy as jnp
from jax.experimental import pallas as pl
from jax.experimental.pallas import tpu as pltpu

_B = 1024
_D = 768
_T = 2
_G = 4
_E = 8
_NCLS = 4
_BLK = 256
_NBLK = 16
_P = _NBLK * _BLK

_bf = jnp.bfloat16
_f32 = jnp.float32


def _b2(x):
    return x.astype(_bf)


def _dot(a, b):
    return jnp.dot(_b2(a), _b2(b), preferred_element_type=_f32)


def _gates_body(n_in, gate_in_ref, gw_ref, gb_ref,
                xin_ref, dw_ref, pos0_ref, pos1_ref, w0_ref, w1_ref,
                be_ref):
    gate_in = gate_in_ref[...]
    acc = gate_in[:, 0:_D]
    for i in range(1, n_in):
        acc = acc + gate_in[:, i * _D:(i + 1) * _D]
    xin_ref[...] = _b2(acc * (1.0 / n_in))

    logits = _dot(gate_in, gw_ref[...]) + gb_ref[...]
    iota_e = jax.lax.broadcasted_iota(jnp.int32, (_B, _E), 1)
    m0 = jnp.max(logits, axis=1, keepdims=True)
    i0 = jnp.min(jnp.where(logits == m0, iota_e, _E), axis=1, keepdims=True)
    sel0 = iota_e == i0
    masked = jnp.where(sel0, -jnp.inf, logits)
    m1 = jnp.max(masked, axis=1, keepdims=True)
    i1 = jnp.min(jnp.where(masked == m1, iota_e, _E), axis=1, keepdims=True)
    sel1 = iota_e == i1
    e1 = jnp.exp(m1 - m0)
    s = 1.0 + e1
    w0 = 1.0 / s
    w1 = e1 / s
    w0_ref[...] = w0
    w1_ref[...] = w1
    dw_ref[...] = jnp.where(sel0, w0, 0.0) + jnp.where(sel1, w1, 0.0)

    onehot = (sel0 | sel1).astype(_f32)
    r = jax.lax.broadcasted_iota(jnp.int32, (_B, _B), 0)
    c = jax.lax.broadcasted_iota(jnp.int32, (_B, _B), 1)
    tri = (c < r).astype(_bf)
    ranks = jnp.dot(tri, _b2(onehot), preferred_element_type=_f32)
    counts = jnp.sum(onehot, axis=0, keepdims=True)
    pc = jnp.ceil(counts * (1.0 / _BLK)) * _BLK
    ue = jax.lax.broadcasted_iota(jnp.int32, (_E, _E), 0)
    uc = jax.lax.broadcasted_iota(jnp.int32, (_E, _E), 1)
    upper = (ue < uc).astype(_bf)
    base = jnp.dot(_b2(pc), upper, preferred_element_type=_f32)
    slot = base + ranks
    pos0_ref[...] = jnp.sum(jnp.where(sel0, slot, 0.0), axis=1,
                            keepdims=True).astype(jnp.int32)
    pos1_ref[...] = jnp.sum(jnp.where(sel1, slot, 0.0), axis=1,
                            keepdims=True).astype(jnp.int32)
    bs = base * (1.0 / _BLK)
    bvec = jax.lax.broadcasted_iota(jnp.int32, (_NBLK, 1), 0).astype(_f32)
    be_ref[...] = (jnp.sum((bs <= bvec).astype(_f32), axis=1, keepdims=True)
                   - 1.0).astype(jnp.int32)


def _gates(gate_in, gw, gb, n_in):
    out_shapes = (
        jax.ShapeDtypeStruct((_B, _D), _bf),
        jax.ShapeDtypeStruct((_B, _E), _f32),
        jax.ShapeDtypeStruct((_B, 1), jnp.int32),
        jax.ShapeDtypeStruct((_B, 1), jnp.int32),
        jax.ShapeDtypeStruct((_B, 1), _f32),
        jax.ShapeDtypeStruct((_B, 1), _f32),
        jax.ShapeDtypeStruct((_NBLK, 1), jnp.int32),
    )
    return pl.pallas_call(
        functools.partial(_gates_body, n_in),
        out_shape=out_shapes,
    )(gate_in, gw, gb.reshape(1, _E))


def _dense_expert_body(xin_ref, dw_ref, w1_ref, b1_ref, w2_ref, b2_ref,
                       w3_ref, b3_ref, w4_ref, b4_ref, acc_ref):
    e = pl.program_id(0)
    x = xin_ref[...]
    h = jax.nn.relu(_dot(x, w1_ref[0]) + b1_ref[0])
    h = jax.nn.relu(_dot(h, w2_ref[0]) + b2_ref[0])
    h = jax.nn.relu(_dot(h, w3_ref[0]) + b3_ref[0])
    y = _dot(h, w4_ref[0]) + b4_ref[0]
    iota_e = jax.lax.broadcasted_iota(jnp.int32, (_B, _E), 1)
    wcol = jnp.sum(jnp.where(iota_e == e, dw_ref[...], 0.0), axis=1,
                   keepdims=True)

    @pl.when(e == 0)
    def _():
        acc_ref[...] = jnp.zeros_like(acc_ref)

    acc_ref[...] += wcol * y


def _dense_experts(xin, dw, mlp, out_dim):
    (w1, b1), (w2, b2), (w3, b3), (w4, b4) = mlp
    h1, h2, h3 = w1.shape[2], w2.shape[2], w3.shape[2]
    full = lambda shape: pl.BlockSpec(shape, lambda e: (0,) * len(shape))
    exp = lambda shape: pl.BlockSpec((1,) + shape, lambda e: (e,) + (0,) * len(shape))
    return pl.pallas_call(
        _dense_expert_body,
        grid=(_E,),
        in_specs=[
            full((_B, _D)), full((_B, _E)),
            exp((_D, h1)), exp((1, h1)),
            exp((h1, h2)), exp((1, h2)),
            exp((h2, h3)), exp((1, h3)),
            exp((h3, out_dim)), exp((1, out_dim)),
        ],
        out_specs=full((_B, out_dim)),
        out_shape=jax.ShapeDtypeStruct((_B, out_dim), _f32),
    )(xin, dw, w1, b1.reshape(_E, 1, h1), w2, b2.reshape(_E, 1, h2),
      w3, b3.reshape(_E, 1, h3), w4, b4.reshape(_E, 1, out_dim))


def _router_dense(gate_in, p, n_in, out_dim):
    xin, dw, _pos0, _pos1, _w0, _w1, _be = _gates(
        gate_in, p["gate_W"], p["gate_b"], n_in)
    return _dense_experts(xin, dw, p["mlp"], out_dim)


def kernel(embeddings, params):
    flat = embeddings.reshape(_B, _G * _T * _D)
    group_outs = []
    for g in range(_G):
        gate_in = flat[:, g * _T * _D:(g + 1) * _T * _D]
        group_outs.append(_router_dense(gate_in, params["groups"][g], _T, _D))
    clf_gate_in = jnp.concatenate(group_outs, axis=1)
    return _router_dense(clf_gate_in, params["clf"], _G, _NCLS)
